# baseline (speedup 1.0000x reference)
import dataclasses
import functools

import jax
import jax.numpy as jnp
from jax import lax
from jax.experimental import pallas as pl
from jax.experimental.pallas import tpu as pltpu
from jax.experimental.pallas import tpu_sc as plsc

_BATCH = 16384
_DIM = 32
_LANES = 16
_NUM_CORES = 2
_NUM_SUBCORES = 16
_NUM_WORKERS = _NUM_CORES * _NUM_SUBCORES
_PER_WORKER = _BATCH // _NUM_WORKERS
_CHUNK = 128
_NUM_CHUNKS = _PER_WORKER // _CHUNK
_NUM_USERS = 1000000


def _make_kernel():
    mesh = plsc.VectorSubcoreMesh(core_axis_name="c", subcore_axis_name="s")
    cp = pltpu.CompilerParams()
    if "needs_layout_passes" in pltpu.CompilerParams.__dataclass_fields__:
        cp = dataclasses.replace(cp, needs_layout_passes=False)
    if "use_tc_tiling_on_sc" in pltpu.CompilerParams.__dataclass_fields__:
        cp = dataclasses.replace(cp, use_tc_tiling_on_sc=False)

    @functools.partial(
        pl.kernel,
        mesh=mesh,
        compiler_params=cp,
        out_type=jax.ShapeDtypeStruct((_BATCH,), jnp.float32),
        scratch_types=[
            pltpu.VMEM((_NUM_CHUNKS, _CHUNK), jnp.int32),
            pltpu.VMEM((_NUM_CHUNKS, _CHUNK), jnp.int32),
            pltpu.VMEM((2, _PER_WORKER), jnp.float32),
            pltpu.VMEM((2, _PER_WORKER), jnp.float32),
            pltpu.VMEM((_PER_WORKER,), jnp.float32),
            pltpu.VMEM((_PER_WORKER,), jnp.float32),
            pltpu.SemaphoreType.DMA,
            pltpu.SemaphoreType.DMA,
        ],
    )
    def k(uid_hbm, mid_hbm, uembt_hbm, membt_hbm, ubias_hbm, mbias_hbm,
          out_hbm, uidx, midx, upl, mpl, acc, mbv, sem, sem2):
        wid = lax.axis_index("s") * _NUM_CORES + lax.axis_index("c")
        base = wid * _PER_WORKER

        pltpu.sync_copy(uid_hbm.at[wid], uidx)
        pltpu.sync_copy(mid_hbm.at[wid], midx)

        def fire(d, buf):
            for j in range(_NUM_CHUNKS):
                sl = pl.ds(j * _CHUNK, _CHUNK)
                plane = pl.ds(d * _NUM_USERS, _NUM_USERS)
                pltpu.async_copy(
                    uembt_hbm.at[plane].at[uidx.at[j]], upl.at[buf].at[sl],
                    sem)
                pltpu.async_copy(
                    membt_hbm.at[plane].at[midx.at[j]], mpl.at[buf].at[sl],
                    sem)

        def drain(d, buf):
            for j in range(_NUM_CHUNKS):
                sl = pl.ds(j * _CHUNK, _CHUNK)
                plane = pl.ds(d * _NUM_USERS, _NUM_USERS)
                pltpu.make_async_copy(
                    uembt_hbm.at[plane].at[uidx.at[j]], upl.at[buf].at[sl],
                    sem).wait()
                pltpu.make_async_copy(
                    membt_hbm.at[plane].at[midx.at[j]], mpl.at[buf].at[sl],
                    sem).wait()

        bias_copies = []
        for j in range(_NUM_CHUNKS):
            sl = pl.ds(j * _CHUNK, _CHUNK)
            bias_copies.append(pltpu.async_copy(
                ubias_hbm.at[uidx.at[j]], acc.at[sl], sem2))
            bias_copies.append(pltpu.async_copy(
                mbias_hbm.at[midx.at[j]], mbv.at[sl], sem2))

        fire(0, 0)

        @pl.loop(0, _DIM)
        def _(d):
            buf = lax.rem(d, 2)
            nbuf = 1 - buf
            drain(d, buf)

            @pl.when(d + 1 < _DIM)
            def _():
                fire(d + 1, nbuf)

            @pl.when(d == 0)
            def _():
                for c in bias_copies:
                    c.wait()
                for i in range(_PER_WORKER // _LANES):
                    sl = pl.ds(i * _LANES, _LANES)
                    acc[sl] = acc[sl] + mbv[sl]

            for i in range(_PER_WORKER // _LANES):
                sl = pl.ds(i * _LANES, _LANES)
                acc[sl] = acc[sl] + upl[buf, sl] * mpl[buf, sl]

        pltpu.sync_copy(acc, out_hbm.at[pl.ds(base, _PER_WORKER)])

    return k


def kernel(user_ids, movie_ids, user_emb, movie_emb, user_bias, movie_bias):
    uids = user_ids.astype(jnp.int32).reshape(_NUM_WORKERS, _NUM_CHUNKS, _CHUNK)
    mids = movie_ids.astype(jnp.int32).reshape(_NUM_WORKERS, _NUM_CHUNKS, _CHUNK)
    ubias = user_bias.reshape(-1)
    mbias = movie_bias.reshape(-1)
    k = _make_kernel()
    return k(uids, mids, user_emb.T.reshape(-1), movie_emb.T.reshape(-1),
             ubias, mbias)


# speedup vs baseline: 5.8077x; 5.8077x over previous
import dataclasses
import functools

import jax
import jax.numpy as jnp
from jax import lax
from jax.experimental import pallas as pl
from jax.experimental.pallas import tpu as pltpu
from jax.experimental.pallas import tpu_sc as plsc

_BATCH = 16384
_DIM = 32
_LANES = 16
_NUM_CORES = 2
_NUM_SUBCORES = 16
_NUM_WORKERS = _NUM_CORES * _NUM_SUBCORES
_PER_WORKER = _BATCH // _NUM_WORKERS
_CHUNK = 128
_NUM_CHUNKS = _PER_WORKER // _CHUNK
_ROWS = 250000
_RW = 128


def _make_kernel():
    mesh = plsc.VectorSubcoreMesh(core_axis_name="c", subcore_axis_name="s")
    cp = pltpu.CompilerParams()
    if "needs_layout_passes" in pltpu.CompilerParams.__dataclass_fields__:
        cp = dataclasses.replace(cp, needs_layout_passes=False)

    @functools.partial(
        pl.kernel,
        mesh=mesh,
        compiler_params=cp,
        out_type=jax.ShapeDtypeStruct((_BATCH,), jnp.float32),
        scratch_types=[
            pltpu.VMEM((_NUM_CHUNKS, _CHUNK), jnp.int32),
            pltpu.VMEM((_NUM_CHUNKS, _CHUNK), jnp.int32),
            pltpu.VMEM((_NUM_CHUNKS, _CHUNK), jnp.int32),
            pltpu.VMEM((_NUM_CHUNKS, _CHUNK), jnp.int32),
            pltpu.VMEM((_NUM_CHUNKS, _CHUNK), jnp.int32),
            pltpu.VMEM((_NUM_CHUNKS, _CHUNK), jnp.int32),
            pltpu.VMEM((2, _CHUNK, _RW), jnp.float32),
            pltpu.VMEM((2, _CHUNK, _RW), jnp.float32),
            pltpu.VMEM((_PER_WORKER,), jnp.float32),
            pltpu.VMEM((_PER_WORKER,), jnp.float32),
            pltpu.VMEM((_PER_WORKER,), jnp.float32),
            pltpu.SemaphoreType.DMA,
            pltpu.SemaphoreType.DMA,
        ],
    )
    def k(uid_hbm, mid_hbm, ue_hbm, me_hbm, ubias_hbm, mbias_hbm,
          out_hbm, uidx, midx, urx, mrx, uq, mq, urows, mrows, ubv, mbv,
          outv, sem, sem2):
        wid = lax.axis_index("s") * _NUM_CORES + lax.axis_index("c")
        base = wid * _PER_WORKER
        lanes = lax.iota(jnp.int32, _LANES)

        pltpu.sync_copy(uid_hbm.at[wid], uidx)
        pltpu.sync_copy(mid_hbm.at[wid], midx)

        bias_copies = []
        for j in range(_NUM_CHUNKS):
            sl = pl.ds(j * _CHUNK, _CHUNK)
            bias_copies.append(pltpu.async_copy(
                ubias_hbm.at[uidx.at[j]], ubv.at[sl], sem2))
            bias_copies.append(pltpu.async_copy(
                mbias_hbm.at[midx.at[j]], mbv.at[sl], sem2))

        for j in range(_NUM_CHUNKS):
            for h in range(_CHUNK // _LANES):
                sl = (j, pl.ds(h * _LANES, _LANES))
                iv = uidx[sl]
                uq[sl] = (iv & 3) * _DIM
                urx[sl] = iv >> 2
                iv = midx[sl]
                mq[sl] = (iv & 3) * _DIM
                mrx[sl] = iv >> 2

        def transfers(c, buf):
            return [
                pltpu.make_async_copy(
                    ue_hbm.at[urx.at[c]], urows.at[buf], sem),
                pltpu.make_async_copy(
                    me_hbm.at[mrx.at[c]], mrows.at[buf], sem),
            ]

        def fire(c, buf):
            for t in transfers(c, buf):
                t.start()

        def drain(c, buf):
            for t in transfers(c, buf):
                t.wait()

        def compute(c, buf):
            @pl.loop(0, _CHUNK // _LANES)
            def _(g):
                qu16 = uq[c, pl.ds(g * _LANES, _LANES)]
                qm16 = mq[c, pl.ds(g * _LANES, _LANES)]
                acc = jnp.zeros((_LANES,), jnp.float32)
                for j in range(_LANES):
                    r = g * _LANES + j
                    qa = qu16[j]
                    qb = qm16[j]
                    s = (urows[buf, r, pl.ds(qa, _LANES)]
                         * mrows[buf, r, pl.ds(qb, _LANES)]
                         + urows[buf, r, pl.ds(qa + _LANES, _LANES)]
                         * mrows[buf, r, pl.ds(qb + _LANES, _LANES)])
                    acc = jnp.where(lanes == j, jnp.sum(s), acc)
                osl = pl.ds(c * _CHUNK + g * _LANES, _LANES)
                outv[osl] = acc

        fire(0, 0)
        for c in range(_NUM_CHUNKS):
            if c + 1 < _NUM_CHUNKS:
                fire(c + 1, (c + 1) % 2)
            drain(c, c % 2)
            compute(c, c % 2)

        for bc in bias_copies:
            bc.wait()
        for h in range(_PER_WORKER // _LANES):
            sl = pl.ds(h * _LANES, _LANES)
            outv[sl] = outv[sl] + ubv[sl] + mbv[sl]

        pltpu.sync_copy(outv, out_hbm.at[pl.ds(base, _PER_WORKER)])

    return k


def kernel(user_ids, movie_ids, user_emb, movie_emb, user_bias, movie_bias):
    uids = user_ids.astype(jnp.int32).reshape(_NUM_WORKERS, _NUM_CHUNKS, _CHUNK)
    mids = movie_ids.astype(jnp.int32).reshape(_NUM_WORKERS, _NUM_CHUNKS, _CHUNK)
    ue = user_emb.reshape(_ROWS, _RW)
    me = movie_emb.reshape(_ROWS, _RW)
    ubias = user_bias.reshape(-1)
    mbias = movie_bias.reshape(-1)
    k = _make_kernel()
    return k(uids, mids, ue, me, ubias, mbias)
